# Initial kernel scaffold; baseline (speedup 1.0000x reference)
import functools

import jax
import jax.numpy as jnp
from jax import lax
from jax.experimental import pallas as pl
from jax.experimental.pallas import tpu as pltpu
from jax.experimental.pallas import tpu_sc as plsc

N_EDGES = 6400000
N_SEG = 100000
LANES = 128
N_ROWS = N_EDGES // LANES
NC, NS = 2, 16
NW = NC * NS
ROWS_PER_TILE = N_ROWS // NW
EXTRA_ROWS = N_ROWS - ROWS_PER_TILE * NW
CHUNK = 11
N_CHUNKS = ROWS_PER_TILE // CHUNK
ACC_PAD = 102400
SLICE = ACC_PAD // NS

_mesh = plsc.VectorSubcoreMesh(core_axis_name="c", subcore_axis_name="s")


@functools.partial(
    pl.kernel,
    out_type=jax.ShapeDtypeStruct((NC, ACC_PAD), jnp.float32),
    mesh=_mesh,
    scratch_types=[
        pltpu.VMEM((2, CHUNK, LANES), jnp.int32),
        pltpu.VMEM((2, CHUNK, LANES), jnp.float32),
        pltpu.VMEM((SLICE,), jnp.float32),
        pltpu.VMEM_SHARED((ACC_PAD,), jnp.float32),
        pltpu.SemaphoreType.DMA,
        pltpu.SemaphoreType.DMA,
        pltpu.SemaphoreType.DMA,
    ],
)
def _seg_sum_sc(vals_hbm, ids_hbm, out_hbm, idx_b, val_b, zbuf, acc,
                sem_i, sem_v, sem_s):
    c = lax.axis_index("c")
    s = lax.axis_index("s")
    w = c * NS + s

    z16 = jnp.zeros((16,), jnp.float32)

    def _zb(i, carry):
        zbuf[pl.ds(i * 16, 16)] = z16
        return carry

    lax.fori_loop(0, SLICE // 16, _zb, 0, unroll=8)
    pltpu.sync_copy(zbuf, acc.at[pl.ds(s * SLICE, SLICE)])
    plsc.subcore_barrier()

    base = w * ROWS_PER_TILE

    def _stage(ci, b):
        row0 = base + ci * CHUNK
        pltpu.async_copy(ids_hbm.at[pl.ds(row0, CHUNK)], idx_b.at[b], sem_i)
        pltpu.async_copy(vals_hbm.at[pl.ds(row0, CHUNK)], val_b.at[b], sem_v)

    def _wait_stage(b):
        pltpu.make_async_copy(
            ids_hbm.at[pl.ds(0, CHUNK)], idx_b.at[b], sem_i).wait()
        pltpu.make_async_copy(
            vals_hbm.at[pl.ds(0, CHUNK)], val_b.at[b], sem_v).wait()

    _stage(0, 0)
    _stage(1, 1)

    def _outer(ci0, carry):
        for b in range(2):
            ci = ci0 * 2 + b
            _wait_stage(b)
            descs = [
                pltpu.async_copy(
                    val_b.at[b, j], acc.at[idx_b.at[b, j]], sem_s, add=True)
                for j in range(CHUNK)
            ]
            for d in descs:
                d.wait()

            @pl.when(ci + 2 < N_CHUNKS)
            def _():
                _stage(ci + 2, b)
        return carry

    lax.fori_loop(0, N_CHUNKS // 2, _outer, 0)

    @pl.when(c == 0)
    def _():
        row = NW * ROWS_PER_TILE + s
        pltpu.async_copy(ids_hbm.at[pl.ds(row, 1)],
                         idx_b.at[0, pl.ds(0, 1)], sem_i)
        pltpu.async_copy(vals_hbm.at[pl.ds(row, 1)],
                         val_b.at[0, pl.ds(0, 1)], sem_v)
        pltpu.make_async_copy(ids_hbm.at[pl.ds(0, 1)],
                              idx_b.at[0, pl.ds(0, 1)], sem_i).wait()
        pltpu.make_async_copy(vals_hbm.at[pl.ds(0, 1)],
                              val_b.at[0, pl.ds(0, 1)], sem_v).wait()
        pltpu.async_copy(
            val_b.at[0, 0], acc.at[idx_b.at[0, 0]], sem_s, add=True).wait()

    plsc.subcore_barrier()

    pltpu.sync_copy(acc.at[pl.ds(s * SLICE, SLICE)],
                    out_hbm.at[c, pl.ds(s * SLICE, SLICE)])


def _combine_body(p_ref, o_ref):
    o_ref[...] = jnp.sum(p_ref[...], axis=0, keepdims=True)


def kernel(arg0_1, arg1_1):
    vals = arg0_1.reshape(N_ROWS, LANES)
    ids = arg1_1.astype(jnp.int32).reshape(N_ROWS, LANES)
    partials = _seg_sum_sc(vals, ids)
    summed = pl.pallas_call(
        _combine_body,
        out_shape=jax.ShapeDtypeStruct((1, ACC_PAD), jnp.float32),
    )(partials)
    return (summed[0, :N_SEG].reshape(N_SEG, 1),)


# speedup vs baseline: 30.0893x; 30.0893x over previous
import functools

import jax
import jax.numpy as jnp
from jax import lax
from jax.experimental import pallas as pl
from jax.experimental.pallas import tpu as pltpu
from jax.experimental.pallas import tpu_sc as plsc

N_EDGES = 6400000
N_SEG = 100000
LANES = 128
N_ROWS = N_EDGES // LANES
NC, NS = 2, 16
NW = NC * NS
ROWS_PER_TILE = N_ROWS // NW
EXTRA_ROWS = N_ROWS - ROWS_PER_TILE * NW
CHUNK = 11
N_CHUNKS = ROWS_PER_TILE // CHUNK
ACC_PAD = 102400
SLICE = ACC_PAD // NS

_mesh = plsc.VectorSubcoreMesh(core_axis_name="c", subcore_axis_name="s")


@functools.partial(
    pl.kernel,
    out_type=jax.ShapeDtypeStruct((NC, ACC_PAD), jnp.float32),
    mesh=_mesh,
    scratch_types=[
        pltpu.VMEM((2, CHUNK, LANES), jnp.int32),
        pltpu.VMEM((2, CHUNK, LANES), jnp.float32),
        pltpu.VMEM((SLICE,), jnp.float32),
        pltpu.VMEM_SHARED((ACC_PAD,), jnp.float32),
        pltpu.SemaphoreType.DMA,
        pltpu.SemaphoreType.DMA,
        pltpu.SemaphoreType.DMA,
    ],
    compiler_params=pltpu.CompilerParams(use_tc_tiling_on_sc=False),
)
def _seg_sum_sc(vals_hbm, ids_hbm, out_hbm, idx_b, val_b, zbuf, acc,
                sem_i, sem_v, sem_s):
    c = lax.axis_index("c")
    s = lax.axis_index("s")
    w = c * NS + s

    z16 = jnp.zeros((16,), jnp.float32)

    def _zb(i, carry):
        zbuf[pl.ds(i * 16, 16)] = z16
        return carry

    lax.fori_loop(0, SLICE // 16, _zb, 0, unroll=8)
    pltpu.sync_copy(zbuf, acc.at[pl.ds(s * SLICE, SLICE)])
    plsc.subcore_barrier()

    base = w * ROWS_PER_TILE

    def _stage(ci, b):
        row0 = base + ci * CHUNK
        pltpu.async_copy(ids_hbm.at[pl.ds(row0, CHUNK)], idx_b.at[b], sem_i)
        pltpu.async_copy(vals_hbm.at[pl.ds(row0, CHUNK)], val_b.at[b], sem_v)

    def _wait_stage(b):
        pltpu.make_async_copy(
            ids_hbm.at[pl.ds(0, CHUNK)], idx_b.at[b], sem_i).wait()
        pltpu.make_async_copy(
            vals_hbm.at[pl.ds(0, CHUNK)], val_b.at[b], sem_v).wait()

    _stage(0, 0)
    _stage(1, 1)

    def _outer(ci0, carry):
        for b in range(2):
            ci = ci0 * 2 + b
            _wait_stage(b)
            descs = [
                pltpu.async_copy(
                    val_b.at[b, j], acc.at[idx_b.at[b, j]], sem_s, add=True)
                for j in range(CHUNK)
            ]
            for d in descs:
                d.wait()

            @pl.when(ci + 2 < N_CHUNKS)
            def _():
                _stage(ci + 2, b)
        return carry

    lax.fori_loop(0, N_CHUNKS // 2, _outer, 0)

    @pl.when(c == 0)
    def _():
        row = NW * ROWS_PER_TILE + s
        pltpu.async_copy(ids_hbm.at[pl.ds(row, 1)],
                         idx_b.at[0, pl.ds(0, 1)], sem_i)
        pltpu.async_copy(vals_hbm.at[pl.ds(row, 1)],
                         val_b.at[0, pl.ds(0, 1)], sem_v)
        pltpu.make_async_copy(ids_hbm.at[pl.ds(0, 1)],
                              idx_b.at[0, pl.ds(0, 1)], sem_i).wait()
        pltpu.make_async_copy(vals_hbm.at[pl.ds(0, 1)],
                              val_b.at[0, pl.ds(0, 1)], sem_v).wait()
        pltpu.async_copy(
            val_b.at[0, 0], acc.at[idx_b.at[0, 0]], sem_s, add=True).wait()

    plsc.subcore_barrier()

    pltpu.sync_copy(acc.at[pl.ds(s * SLICE, SLICE)],
                    out_hbm.at[c, pl.ds(s * SLICE, SLICE)])


def _combine_body(p_ref, o_ref):
    o_ref[...] = jnp.sum(p_ref[...], axis=0, keepdims=True)


def kernel(arg0_1, arg1_1):
    vals = arg0_1.reshape(N_ROWS, LANES)
    ids = arg1_1.astype(jnp.int32).reshape(N_ROWS, LANES)
    partials = _seg_sum_sc(vals, ids)
    summed = pl.pallas_call(
        _combine_body,
        out_shape=jax.ShapeDtypeStruct((1, ACC_PAD), jnp.float32),
    )(partials)
    return (summed[0, :N_SEG].reshape(N_SEG, 1),)
